# Initial kernel scaffold; baseline (speedup 1.0000x reference)
import jax
import jax.numpy as jnp
from jax.experimental import pallas as pl


def kernel(x, edge_index, W1, b1, We, be, Wmu, bmu, Wsig, bsig, rk_lgt):
    raise NotImplementedError("write your pallas kernel here")


# speedup vs baseline: 82.2836x; 82.2836x over previous
import functools

import jax
import jax.numpy as jnp
from jax import lax
from jax.experimental import pallas as pl
from jax.experimental.pallas import tpu as pltpu
from jax.experimental.pallas import tpu_sc as plsc

N = 4096
E = 65536
IN_CH = 4096
HID = 128
OUT = 32
E_CH = 32
REWEIGHT = ((E_CH + HID) / (IN_CH + HID)) ** 0.5

ROWS = 512
EPB = 128
BLK = 16


def _mesh():
    return plsc.VectorSubcoreMesh(
        core_axis_name="c", subcore_axis_name="s", num_cores=2, num_subcores=16
    )


def _fill_rows(ref, rows, cols, val):
    vec = jnp.full((16,), val, jnp.float32)

    def body(i, carry):
        for k in range(cols // 16):
            ref[i, pl.ds(k * 16, 16)] = vec
        return carry

    lax.fori_loop(0, rows, body, 0)


@functools.partial(
    pl.kernel,
    out_type=jax.ShapeDtypeStruct((2, N, 16), jnp.float32),
    mesh=_mesh(),
    scratch_types=[
        pltpu.VMEM((BLK, EPB), jnp.int32),
        pltpu.VMEM((EPB, 16), jnp.float32),
        pltpu.VMEM((256, 16), jnp.float32),
        pltpu.VMEM_SHARED((N, 16), jnp.float32),
    ],
)
def _hist_k(dst_hbm, out_hbm, idx_v, ones_v, zbuf, acc):
    c = lax.axis_index("c")
    s = lax.axis_index("s")
    w = s * 2 + c
    _fill_rows(zbuf, 256, 16, 0.0)
    _fill_rows(ones_v, EPB, 16, 1.0)
    pltpu.sync_copy(zbuf, acc.at[pl.ds(s * 256, 256)])
    plsc.subcore_barrier()
    pltpu.sync_copy(dst_hbm.at[pl.ds(w * BLK, BLK)], idx_v)
    for j in range(BLK):
        pltpu.sync_copy(ones_v, acc.at[idx_v.at[j]], add=True)
    plsc.subcore_barrier()
    pltpu.sync_copy(acc.at[pl.ds(s * 256, 256)], out_hbm.at[c, pl.ds(s * 256, 256)])


def _make_agg(C):
    @functools.partial(
        pl.kernel,
        out_type=jax.ShapeDtypeStruct((2, N, C), jnp.float32),
        mesh=_mesh(),
        scratch_types=[
            pltpu.VMEM((BLK, EPB), jnp.int32),
            pltpu.VMEM((BLK, EPB), jnp.int32),
            pltpu.VMEM((EPB, C), jnp.float32),
            pltpu.VMEM_SHARED((N, C), jnp.float32),
            pltpu.SemaphoreType.DMA,
        ],
    )
    def agg_k(src_hbm, dst_hbm, feat_hbm, out_hbm, src_v, dst_v, gbuf, acc, sem):
        c = lax.axis_index("c")
        s = lax.axis_index("s")
        w = s * 2 + c
        _fill_rows(gbuf, EPB, C, 0.0)
        pltpu.sync_copy(gbuf, acc.at[pl.ds(s * 256, 128)])
        pltpu.sync_copy(gbuf, acc.at[pl.ds(s * 256 + 128, 128)])
        plsc.subcore_barrier()
        pltpu.sync_copy(src_hbm.at[pl.ds(w * BLK, BLK)], src_v)
        pltpu.sync_copy(dst_hbm.at[pl.ds(w * BLK, BLK)], dst_v)
        for j in range(BLK):
            pltpu.async_copy(feat_hbm.at[src_v.at[j]], gbuf, sem).wait()
            pltpu.sync_copy(gbuf, acc.at[dst_v.at[j]], add=True)
        plsc.subcore_barrier()
        pltpu.sync_copy(acc.at[pl.ds(s * 256, 128)], out_hbm.at[c, pl.ds(s * 256, 128)])
        pltpu.sync_copy(
            acc.at[pl.ds(s * 256 + 128, 128)], out_hbm.at[c, pl.ds(s * 256 + 128, 128)]
        )

    return agg_k


_agg128 = _make_agg(128)


def _dinv_of(degp_ref):
    deg = degp_ref[0] + degp_ref[1]
    return lax.rsqrt(deg[:, :1] + 1.0)


def _p1_body(x_ref, w1_ref, e2_ref, web_ref, degp_ref, hx_ref, he0_ref, he1_ref):
    dinv = _dinv_of(degp_ref)
    h1 = jnp.dot(x_ref[...], w1_ref[...], preferred_element_type=jnp.float32)
    he = jnp.dot(e2_ref[...], web_ref[...], preferred_element_type=jnp.float32)
    hx_ref[...] = h1 * dinv
    he0_ref[...] = he[:, :128] * dinv
    he1_ref[...] = he[:, 128:] * dinv


def _p1(x, W1, e2, web, degp):
    return pl.pallas_call(
        _p1_body,
        grid=(N // ROWS,),
        in_specs=[
            pl.BlockSpec((ROWS, IN_CH), lambda i: (i, 0)),
            pl.BlockSpec((IN_CH, HID), lambda i: (0, 0)),
            pl.BlockSpec((ROWS, 64), lambda i: (i, 0)),
            pl.BlockSpec((64, 256), lambda i: (0, 0)),
            pl.BlockSpec((2, ROWS, 16), lambda i: (0, i, 0)),
        ],
        out_specs=[
            pl.BlockSpec((ROWS, 128), lambda i: (i, 0)),
            pl.BlockSpec((ROWS, 128), lambda i: (i, 0)),
            pl.BlockSpec((ROWS, 128), lambda i: (i, 0)),
        ],
        out_shape=[
            jax.ShapeDtypeStruct((N, 128), jnp.float32),
            jax.ShapeDtypeStruct((N, 128), jnp.float32),
            jax.ShapeDtypeStruct((N, 128), jnp.float32),
        ],
    )(x, W1, e2, web, degp)


def _p4_body(rx_ref, re0_ref, re1_ref, hxs_ref, he0s_ref, he1s_ref, degp_ref,
             b1_ref, be_ref, w4_ref, hm2_ref, sums_ref):
    dinv = _dinv_of(degp_ref)
    hx = jnp.maximum((rx_ref[0] + rx_ref[1] + hxs_ref[...]) * dinv + b1_ref[...], 0.0)
    he0 = (re0_ref[0] + re0_ref[1] + he0s_ref[...]) * dinv + be_ref[...]
    he1 = (re1_ref[0] + re1_ref[1] + he1s_ref[...]) * dinv + be_ref[...]
    hcat = jnp.concatenate([hx + he0, hx + he1], axis=1)
    hm2_ref[...] = (
        jnp.dot(hcat, w4_ref[...], preferred_element_type=jnp.float32) * dinv
    )
    part = jnp.stack(
        [
            jnp.sum(hx * hx, axis=0),
            jnp.sum(he0 * he0, axis=0),
            jnp.sum(he1 * he1, axis=0),
        ]
    )

    @pl.when(pl.program_id(0) == 0)
    def _init():
        sums_ref[...] = jnp.zeros_like(sums_ref)

    sums_ref[...] += part


def _p4(rx, re0, re1, hxs, he0s, he1s, degp, b1r, ber, w4):
    return pl.pallas_call(
        _p4_body,
        grid=(N // ROWS,),
        in_specs=[
            pl.BlockSpec((2, ROWS, 128), lambda i: (0, i, 0)),
            pl.BlockSpec((2, ROWS, 128), lambda i: (0, i, 0)),
            pl.BlockSpec((2, ROWS, 128), lambda i: (0, i, 0)),
            pl.BlockSpec((ROWS, 128), lambda i: (i, 0)),
            pl.BlockSpec((ROWS, 128), lambda i: (i, 0)),
            pl.BlockSpec((ROWS, 128), lambda i: (i, 0)),
            pl.BlockSpec((2, ROWS, 16), lambda i: (0, i, 0)),
            pl.BlockSpec((1, HID), lambda i: (0, 0)),
            pl.BlockSpec((1, HID), lambda i: (0, 0)),
            pl.BlockSpec((256, 128), lambda i: (0, 0)),
        ],
        out_specs=[
            pl.BlockSpec((ROWS, 128), lambda i: (i, 0)),
            pl.BlockSpec((3, 128), lambda i: (0, 0)),
        ],
        out_shape=[
            jax.ShapeDtypeStruct((N, 128), jnp.float32),
            jax.ShapeDtypeStruct((3, 128), jnp.float32),
        ],
    )(rx, re0, re1, hxs, he0s, he1s, degp, b1r, ber, w4)


def _p6_body(r2_ref, hm2_ref, degp_ref, bms_ref, eps_ref, rk_ref, ms_ref, z_ref, rk2_ref):
    dinv = _dinv_of(degp_ref)
    ms = (r2_ref[0] + r2_ref[1] + hm2_ref[...]) * dinv + bms_ref[...]
    ms_ref[...] = ms
    z_ref[...] = eps_ref[...] * jnp.exp(ms[:, 96:128] * 0.5) + ms[:, 32:64]

    @pl.when(pl.program_id(0) == 0)
    def _rk():
        rk2_ref[...] = jax.nn.sigmoid(rk_ref[...])


def _p6(r2, hm2, degp, bms, eps0, rk_lgt):
    return pl.pallas_call(
        _p6_body,
        grid=(N // ROWS,),
        in_specs=[
            pl.BlockSpec((2, ROWS, 128), lambda i: (0, i, 0)),
            pl.BlockSpec((ROWS, 128), lambda i: (i, 0)),
            pl.BlockSpec((2, ROWS, 16), lambda i: (0, i, 0)),
            pl.BlockSpec((1, 128), lambda i: (0, 0)),
            pl.BlockSpec((ROWS, OUT), lambda i: (i, 0)),
            pl.BlockSpec((1, OUT), lambda i: (0, 0)),
        ],
        out_specs=[
            pl.BlockSpec((ROWS, 128), lambda i: (i, 0)),
            pl.BlockSpec((ROWS, OUT), lambda i: (i, 0)),
            pl.BlockSpec((1, OUT), lambda i: (0, 0)),
        ],
        out_shape=[
            jax.ShapeDtypeStruct((N, 128), jnp.float32),
            jax.ShapeDtypeStruct((N, OUT), jnp.float32),
            jax.ShapeDtypeStruct((1, OUT), jnp.float32),
        ],
    )(r2, hm2, degp, bms, eps0, rk_lgt)


def _p7_body(zb_ref, zf_ref, adj_ref):
    prod = lax.dot_general(
        zb_ref[...], zf_ref[...], (((1,), (1,)), ((), ())),
        preferred_element_type=jnp.float32,
    )
    adj_ref[...] = jax.nn.sigmoid(prod)


def _p7(z):
    return pl.pallas_call(
        _p7_body,
        grid=(N // ROWS,),
        in_specs=[
            pl.BlockSpec((ROWS, OUT), lambda i: (i, 0)),
            pl.BlockSpec((N, OUT), lambda i: (0, 0)),
        ],
        out_specs=pl.BlockSpec((ROWS, N), lambda i: (i, 0)),
        out_shape=jax.ShapeDtypeStruct((N, N), jnp.float32),
    )(z, z)


def kernel(x, edge_index, W1, b1, We, be, Wmu, bmu, Wsig, bsig, rk_lgt):
    f32 = jnp.float32
    src = edge_index[0].astype(jnp.int32).reshape(E // EPB, EPB)
    dst = edge_index[1].astype(jnp.int32).reshape(E // EPB, EPB)

    e = jax.random.normal(jax.random.key(42), (2, N, E_CH), f32) * REWEIGHT
    eps = jax.random.normal(jax.random.key(7), (1, N, OUT), f32)
    e2 = jnp.concatenate([e[0], e[1]], axis=1)
    web = (
        jnp.zeros((64, 256), f32).at[:32, :128].set(We).at[32:, 128:].set(We)
    )
    w4 = (
        jnp.zeros((256, 128), f32)
        .at[:128, :32].set(Wmu)
        .at[128:, 32:64].set(Wmu)
        .at[:128, 64:96].set(Wsig)
        .at[128:, 96:].set(Wsig)
    )
    bms = jnp.concatenate([bmu, bmu, bsig, bsig]).reshape(1, 128)
    b1r = b1.reshape(1, HID)
    ber = be.reshape(1, HID)

    degp = _hist_k(dst)
    hxs, he0s, he1s = _p1(x, W1, e2, web, degp)
    rx = _agg128(src, dst, hxs)
    re0 = _agg128(src, dst, he0s)
    re1 = _agg128(src, dst, he1s)
    hm2, sums = _p4(rx, re0, re1, hxs, he0s, he1s, degp, b1r, ber, w4)
    r2 = _agg128(src, dst, hm2)
    ms, z, rk2 = _p6(r2, hm2, degp, bms, eps[0], rk_lgt)
    adj = _p7(z)

    denom = jnp.float32(N * HID)
    p_signal = jnp.sum(sums[0]) / denom
    p_noise = jnp.stack([jnp.sum(sums[1]), jnp.sum(sums[2])]) / denom
    snr = p_signal / p_noise
    mu = jnp.stack([ms[:, :32], ms[:, 32:64]])
    sigma = jnp.stack([ms[:, 64:96], ms[:, 96:128]])
    z3 = z[None]
    return adj[None], mu, sigma, z3, z3, eps, rk2, snr


# speedup vs baseline: 105.9328x; 1.2874x over previous
import functools

import jax
import jax.numpy as jnp
from jax import lax
from jax.experimental import pallas as pl
from jax.experimental.pallas import tpu as pltpu
from jax.experimental.pallas import tpu_sc as plsc

N = 4096
E = 65536
IN_CH = 4096
HID = 128
OUT = 32
E_CH = 32
REWEIGHT = ((E_CH + HID) / (IN_CH + HID)) ** 0.5

ROWS = 512
EPB = 128
BLK = 16


def _mesh():
    return plsc.VectorSubcoreMesh(
        core_axis_name="c", subcore_axis_name="s", num_cores=2, num_subcores=16
    )


def _fill_rows(ref, rows, cols, val):
    vec = jnp.full((16,), val, jnp.float32)

    def body(i, carry):
        for k in range(cols // 16):
            ref[i, pl.ds(k * 16, 16)] = vec
        return carry

    lax.fori_loop(0, rows, body, 0)


@functools.partial(
    pl.kernel,
    out_type=jax.ShapeDtypeStruct((2, N, 16), jnp.float32),
    mesh=_mesh(),
    scratch_types=[
        pltpu.VMEM((BLK, EPB), jnp.int32),
        pltpu.VMEM((EPB, 16), jnp.float32),
        pltpu.VMEM((256, 16), jnp.float32),
        pltpu.VMEM_SHARED((N, 16), jnp.float32),
    ],
)
def _hist_k(dst_hbm, out_hbm, idx_v, ones_v, zbuf, acc):
    c = lax.axis_index("c")
    s = lax.axis_index("s")
    w = s * 2 + c
    _fill_rows(zbuf, 256, 16, 0.0)
    _fill_rows(ones_v, EPB, 16, 1.0)
    pltpu.sync_copy(zbuf, acc.at[pl.ds(s * 256, 256)])
    plsc.subcore_barrier()
    pltpu.sync_copy(dst_hbm.at[pl.ds(w * BLK, BLK)], idx_v)
    for j in range(BLK):
        pltpu.sync_copy(ones_v, acc.at[idx_v.at[j]], add=True)
    plsc.subcore_barrier()
    pltpu.sync_copy(acc.at[pl.ds(s * 256, 256)], out_hbm.at[c, pl.ds(s * 256, 256)])


def _make_agg(n_feats):
    out_types = [jax.ShapeDtypeStruct((2, N, 128), jnp.float32)] * n_feats

    @functools.partial(
        pl.kernel,
        out_type=out_types,
        mesh=_mesh(),
        scratch_types=[
            pltpu.VMEM((BLK, EPB), jnp.int32),
            pltpu.VMEM((BLK, EPB), jnp.int32),
            pltpu.VMEM((2, EPB, 128), jnp.float32),
            pltpu.VMEM((128, 128), jnp.float32),
            pltpu.VMEM_SHARED((N, 128), jnp.float32),
            pltpu.SemaphoreType.DMA,
            pltpu.SemaphoreType.DMA,
        ],
    )
    def agg_k(src_hbm, dst_hbm, *rest):
        feats = rest[:n_feats]
        outs = rest[n_feats:2 * n_feats]
        src_v, dst_v, gbuf, zbuf, acc, sem0, sem1 = rest[2 * n_feats:]
        c = lax.axis_index("c")
        s = lax.axis_index("s")
        w = s * 2 + c
        sems = (sem0, sem1)

        _fill_rows(zbuf, 128, 128, 0.0)
        pltpu.sync_copy(zbuf, acc.at[pl.ds(s * 256, 128)])
        pltpu.sync_copy(zbuf, acc.at[pl.ds(s * 256 + 128, 128)])
        pltpu.sync_copy(src_hbm.at[pl.ds(w * BLK, BLK)], src_v)
        pltpu.sync_copy(dst_hbm.at[pl.ds(w * BLK, BLK)], dst_v)
        plsc.subcore_barrier()

        for f in range(n_feats):
            feat = feats[f]
            descs = [None, None]
            descs[0] = pltpu.async_copy(feat.at[src_v.at[0]], gbuf.at[0], sems[0])
            for j in range(BLK):
                nxt = j + 1
                if nxt < BLK:
                    descs[nxt % 2] = pltpu.async_copy(
                        feat.at[src_v.at[nxt]], gbuf.at[nxt % 2], sems[nxt % 2]
                    )
                descs[j % 2].wait()
                pltpu.sync_copy(gbuf.at[j % 2], acc.at[dst_v.at[j]], add=True)
            plsc.subcore_barrier()
            out = outs[f]
            pltpu.sync_copy(acc.at[pl.ds(s * 256, 128)], out.at[c, pl.ds(s * 256, 128)])
            pltpu.sync_copy(
                acc.at[pl.ds(s * 256 + 128, 128)], out.at[c, pl.ds(s * 256 + 128, 128)]
            )
            if f + 1 < n_feats:
                pltpu.sync_copy(zbuf, acc.at[pl.ds(s * 256, 128)])
                pltpu.sync_copy(zbuf, acc.at[pl.ds(s * 256 + 128, 128)])
                plsc.subcore_barrier()

    return agg_k


_agg3 = _make_agg(3)
_agg1 = _make_agg(1)


def _dinv_of(degp_ref):
    deg = degp_ref[0] + degp_ref[1]
    return lax.rsqrt(deg[:, :1] + 1.0)


def _p1_body(x_ref, w1_ref, e2_ref, web_ref, degp_ref, hx_ref, he0_ref, he1_ref):
    dinv = _dinv_of(degp_ref)
    h1 = jnp.dot(x_ref[...], w1_ref[...], preferred_element_type=jnp.float32)
    he = jnp.dot(e2_ref[...], web_ref[...], preferred_element_type=jnp.float32)
    hx_ref[...] = h1 * dinv
    he0_ref[...] = he[:, :128] * dinv
    he1_ref[...] = he[:, 128:] * dinv


def _p1(x, W1, e2, web, degp):
    return pl.pallas_call(
        _p1_body,
        grid=(N // ROWS,),
        in_specs=[
            pl.BlockSpec((ROWS, IN_CH), lambda i: (i, 0)),
            pl.BlockSpec((IN_CH, HID), lambda i: (0, 0)),
            pl.BlockSpec((ROWS, 64), lambda i: (i, 0)),
            pl.BlockSpec((64, 256), lambda i: (0, 0)),
            pl.BlockSpec((2, ROWS, 16), lambda i: (0, i, 0)),
        ],
        out_specs=[
            pl.BlockSpec((ROWS, 128), lambda i: (i, 0)),
            pl.BlockSpec((ROWS, 128), lambda i: (i, 0)),
            pl.BlockSpec((ROWS, 128), lambda i: (i, 0)),
        ],
        out_shape=[
            jax.ShapeDtypeStruct((N, 128), jnp.float32),
            jax.ShapeDtypeStruct((N, 128), jnp.float32),
            jax.ShapeDtypeStruct((N, 128), jnp.float32),
        ],
    )(x, W1, e2, web, degp)


def _p4_body(rx_ref, re0_ref, re1_ref, hxs_ref, he0s_ref, he1s_ref, degp_ref,
             b1_ref, be_ref, w4_ref, hm2_ref, sums_ref, snr_ref):
    dinv = _dinv_of(degp_ref)
    hx = jnp.maximum((rx_ref[0] + rx_ref[1] + hxs_ref[...]) * dinv + b1_ref[...], 0.0)
    he0 = (re0_ref[0] + re0_ref[1] + he0s_ref[...]) * dinv + be_ref[...]
    he1 = (re1_ref[0] + re1_ref[1] + he1s_ref[...]) * dinv + be_ref[...]
    hcat = jnp.concatenate([hx + he0, hx + he1], axis=1)
    hm2_ref[...] = (
        jnp.dot(hcat, w4_ref[...], preferred_element_type=jnp.float32) * dinv
    )
    part = jnp.stack(
        [
            jnp.sum(hx * hx, axis=0),
            jnp.sum(he0 * he0, axis=0),
            jnp.sum(he1 * he1, axis=0),
        ]
    )

    @pl.when(pl.program_id(0) == 0)
    def _init():
        sums_ref[...] = jnp.zeros_like(sums_ref)

    sums_ref[...] += part

    @pl.when(pl.program_id(0) == pl.num_programs(0) - 1)
    def _snr():
        t = jnp.sum(sums_ref[...], axis=1)
        snr_ref[...] = jnp.broadcast_to(
            (t[0] / jnp.stack([t[1], t[2]]))[:, None], (2, 128)
        )


def _p4(rx, re0, re1, hxs, he0s, he1s, degp, b1r, ber, w4):
    return pl.pallas_call(
        _p4_body,
        grid=(N // ROWS,),
        in_specs=[
            pl.BlockSpec((2, ROWS, 128), lambda i: (0, i, 0)),
            pl.BlockSpec((2, ROWS, 128), lambda i: (0, i, 0)),
            pl.BlockSpec((2, ROWS, 128), lambda i: (0, i, 0)),
            pl.BlockSpec((ROWS, 128), lambda i: (i, 0)),
            pl.BlockSpec((ROWS, 128), lambda i: (i, 0)),
            pl.BlockSpec((ROWS, 128), lambda i: (i, 0)),
            pl.BlockSpec((2, ROWS, 16), lambda i: (0, i, 0)),
            pl.BlockSpec((1, HID), lambda i: (0, 0)),
            pl.BlockSpec((1, HID), lambda i: (0, 0)),
            pl.BlockSpec((256, 128), lambda i: (0, 0)),
        ],
        out_specs=[
            pl.BlockSpec((ROWS, 128), lambda i: (i, 0)),
            pl.BlockSpec((3, 128), lambda i: (0, 0)),
            pl.BlockSpec((2, 128), lambda i: (0, 0)),
        ],
        out_shape=[
            jax.ShapeDtypeStruct((N, 128), jnp.float32),
            jax.ShapeDtypeStruct((3, 128), jnp.float32),
            jax.ShapeDtypeStruct((2, 128), jnp.float32),
        ],
    )(rx, re0, re1, hxs, he0s, he1s, degp, b1r, ber, w4)


def _p6_body(r2_ref, hm2_ref, degp_ref, bms_ref, eps_ref, rk_ref,
             mu_ref, sig_ref, z_ref, rk2_ref):
    dinv = _dinv_of(degp_ref)
    ms = (r2_ref[0] + r2_ref[1] + hm2_ref[...]) * dinv + bms_ref[...]
    mu_ref[0] = ms[:, :32]
    mu_ref[1] = ms[:, 32:64]
    sig_ref[0] = ms[:, 64:96]
    sig_ref[1] = ms[:, 96:128]
    z_ref[...] = eps_ref[...] * jnp.exp(ms[:, 96:128] * 0.5) + ms[:, 32:64]

    @pl.when(pl.program_id(0) == 0)
    def _rk():
        rk2_ref[...] = jax.nn.sigmoid(rk_ref[...])


def _p6(r2, hm2, degp, bms, eps0, rk_lgt):
    return pl.pallas_call(
        _p6_body,
        grid=(N // ROWS,),
        in_specs=[
            pl.BlockSpec((2, ROWS, 128), lambda i: (0, i, 0)),
            pl.BlockSpec((ROWS, 128), lambda i: (i, 0)),
            pl.BlockSpec((2, ROWS, 16), lambda i: (0, i, 0)),
            pl.BlockSpec((1, 128), lambda i: (0, 0)),
            pl.BlockSpec((ROWS, OUT), lambda i: (i, 0)),
            pl.BlockSpec((1, OUT), lambda i: (0, 0)),
        ],
        out_specs=[
            pl.BlockSpec((2, ROWS, OUT), lambda i: (0, i, 0)),
            pl.BlockSpec((2, ROWS, OUT), lambda i: (0, i, 0)),
            pl.BlockSpec((ROWS, OUT), lambda i: (i, 0)),
            pl.BlockSpec((1, OUT), lambda i: (0, 0)),
        ],
        out_shape=[
            jax.ShapeDtypeStruct((2, N, OUT), jnp.float32),
            jax.ShapeDtypeStruct((2, N, OUT), jnp.float32),
            jax.ShapeDtypeStruct((N, OUT), jnp.float32),
            jax.ShapeDtypeStruct((1, OUT), jnp.float32),
        ],
    )(r2, hm2, degp, bms, eps0, rk_lgt)


def _p7_body(zb_ref, zf_ref, adj_ref):
    prod = lax.dot_general(
        zb_ref[...], zf_ref[...], (((1,), (1,)), ((), ())),
        preferred_element_type=jnp.float32,
    )
    adj_ref[...] = jax.nn.sigmoid(prod)


def _p7(z):
    return pl.pallas_call(
        _p7_body,
        grid=(N // ROWS,),
        in_specs=[
            pl.BlockSpec((ROWS, OUT), lambda i: (i, 0)),
            pl.BlockSpec((N, OUT), lambda i: (0, 0)),
        ],
        out_specs=pl.BlockSpec((ROWS, N), lambda i: (i, 0)),
        out_shape=jax.ShapeDtypeStruct((N, N), jnp.float32),
    )(z, z)


def kernel(x, edge_index, W1, b1, We, be, Wmu, bmu, Wsig, bsig, rk_lgt):
    f32 = jnp.float32
    src = edge_index[0].astype(jnp.int32).reshape(E // EPB, EPB)
    dst = edge_index[1].astype(jnp.int32).reshape(E // EPB, EPB)

    e = jax.random.normal(jax.random.key(42), (2, N, E_CH), f32) * REWEIGHT
    eps = jax.random.normal(jax.random.key(7), (1, N, OUT), f32)
    e2 = jnp.concatenate([e[0], e[1]], axis=1)
    web = (
        jnp.zeros((64, 256), f32).at[:32, :128].set(We).at[32:, 128:].set(We)
    )
    w4 = (
        jnp.zeros((256, 128), f32)
        .at[:128, :32].set(Wmu)
        .at[128:, 32:64].set(Wmu)
        .at[:128, 64:96].set(Wsig)
        .at[128:, 96:].set(Wsig)
    )
    bms = jnp.concatenate([bmu, bmu, bsig, bsig]).reshape(1, 128)
    b1r = b1.reshape(1, HID)
    ber = be.reshape(1, HID)

    degp = _hist_k(dst)
    hxs, he0s, he1s = _p1(x, W1, e2, web, degp)
    rx, re0, re1 = _agg3(src, dst, hxs, he0s, he1s)
    hm2, sums, snr2 = _p4(rx, re0, re1, hxs, he0s, he1s, degp, b1r, ber, w4)
    (r2,) = _agg1(src, dst, hm2)
    mu, sigma, z, rk2 = _p6(r2, hm2, degp, bms, eps[0], rk_lgt)
    adj = _p7(z)

    snr = snr2[:, 0]
    z3 = z[None]
    return adj[None], mu, sigma, z3, z3, eps, rk2, snr


# speedup vs baseline: 122.6650x; 1.1580x over previous
import functools

import jax
import jax.numpy as jnp
from jax import lax
from jax.experimental import pallas as pl
from jax.experimental.pallas import tpu as pltpu
from jax.experimental.pallas import tpu_sc as plsc

N = 4096
E = 65536
IN_CH = 4096
HID = 128
OUT = 32
E_CH = 32
REWEIGHT = ((E_CH + HID) / (IN_CH + HID)) ** 0.5

ROWS = 512
EPB = 128
BLK = 16


def _mesh():
    return plsc.VectorSubcoreMesh(
        core_axis_name="c", subcore_axis_name="s", num_cores=2, num_subcores=16
    )


def _fill_rows(ref, rows, cols, val):
    vec = jnp.full((16,), val, jnp.float32)

    def body(i, carry):
        for k in range(cols // 16):
            ref[i, pl.ds(k * 16, 16)] = vec
        return carry

    lax.fori_loop(0, rows, body, 0)


@functools.partial(
    pl.kernel,
    out_type=jax.ShapeDtypeStruct((2, N, 16), jnp.float32),
    mesh=_mesh(),
    scratch_types=[
        pltpu.VMEM((BLK, EPB), jnp.int32),
        pltpu.VMEM((EPB, 16), jnp.float32),
        pltpu.VMEM((256, 16), jnp.float32),
        pltpu.VMEM_SHARED((N, 16), jnp.float32),
    ],
)
def _hist_k(dst_hbm, out_hbm, idx_v, ones_v, zbuf, acc):
    c = lax.axis_index("c")
    s = lax.axis_index("s")
    w = s * 2 + c
    _fill_rows(zbuf, 256, 16, 0.0)
    _fill_rows(ones_v, EPB, 16, 1.0)
    pltpu.sync_copy(zbuf, acc.at[pl.ds(s * 256, 256)])
    plsc.subcore_barrier()
    pltpu.sync_copy(dst_hbm.at[pl.ds(w * BLK, BLK)], idx_v)
    for j in range(BLK):
        pltpu.sync_copy(ones_v, acc.at[idx_v.at[j]], add=True)
    plsc.subcore_barrier()
    pltpu.sync_copy(acc.at[pl.ds(s * 256, 256)], out_hbm.at[c, pl.ds(s * 256, 256)])


def _make_agg(n_feats):
    out_types = [jax.ShapeDtypeStruct((2, N, 128), jnp.float32)] * n_feats

    @functools.partial(
        pl.kernel,
        out_type=out_types,
        mesh=_mesh(),
        scratch_types=[
            pltpu.VMEM((BLK, EPB), jnp.int32),
            pltpu.VMEM((BLK, EPB), jnp.int32),
            pltpu.VMEM((2, EPB, 128), jnp.float32),
            pltpu.VMEM((128, 128), jnp.float32),
            pltpu.VMEM_SHARED((N, 128), jnp.float32),
            pltpu.SemaphoreType.DMA,
            pltpu.SemaphoreType.DMA,
        ],
    )
    def agg_k(src_hbm, dst_hbm, *rest):
        feats = rest[:n_feats]
        outs = rest[n_feats:2 * n_feats]
        src_v, dst_v, gbuf, zbuf, acc, sem0, sem1 = rest[2 * n_feats:]
        c = lax.axis_index("c")
        s = lax.axis_index("s")
        w = s * 2 + c
        sems = (sem0, sem1)

        _fill_rows(zbuf, 128, 128, 0.0)
        pltpu.sync_copy(zbuf, acc.at[pl.ds(s * 256, 128)])
        pltpu.sync_copy(zbuf, acc.at[pl.ds(s * 256 + 128, 128)])
        pltpu.sync_copy(src_hbm.at[pl.ds(w * BLK, BLK)], src_v)
        pltpu.sync_copy(dst_hbm.at[pl.ds(w * BLK, BLK)], dst_v)
        plsc.subcore_barrier()

        for f in range(n_feats):
            feat = feats[f]
            descs = [None, None]
            descs[0] = pltpu.async_copy(feat.at[src_v.at[0]], gbuf.at[0], sems[0])
            for j in range(BLK):
                nxt = j + 1
                if nxt < BLK:
                    descs[nxt % 2] = pltpu.async_copy(
                        feat.at[src_v.at[nxt]], gbuf.at[nxt % 2], sems[nxt % 2]
                    )
                descs[j % 2].wait()
                pltpu.sync_copy(gbuf.at[j % 2], acc.at[dst_v.at[j]], add=True)
            plsc.subcore_barrier()
            out = outs[f]
            pltpu.sync_copy(acc.at[pl.ds(s * 256, 128)], out.at[c, pl.ds(s * 256, 128)])
            pltpu.sync_copy(
                acc.at[pl.ds(s * 256 + 128, 128)], out.at[c, pl.ds(s * 256 + 128, 128)]
            )
            if f + 1 < n_feats:
                pltpu.sync_copy(zbuf, acc.at[pl.ds(s * 256, 128)])
                pltpu.sync_copy(zbuf, acc.at[pl.ds(s * 256 + 128, 128)])
                plsc.subcore_barrier()

    return agg_k


_agg1 = _make_agg(1)


def _dinv_of(degp_ref):
    deg = degp_ref[0] + degp_ref[1]
    return lax.rsqrt(deg[:, :1] + 1.0)


def _p1a_body(x_ref, w1_ref, degp_ref, hx_ref):
    dinv = _dinv_of(degp_ref)
    h1 = jnp.dot(x_ref[...], w1_ref[...], preferred_element_type=jnp.float32)
    hx_ref[...] = h1 * dinv


def _p1a(x, W1, degp):
    return pl.pallas_call(
        _p1a_body,
        grid=(N // ROWS,),
        in_specs=[
            pl.BlockSpec((ROWS, IN_CH), lambda i: (i, 0)),
            pl.BlockSpec((IN_CH, HID), lambda i: (0, 0)),
            pl.BlockSpec((2, ROWS, 16), lambda i: (0, i, 0)),
        ],
        out_specs=pl.BlockSpec((ROWS, 128), lambda i: (i, 0)),
        out_shape=jax.ShapeDtypeStruct((N, 128), jnp.float32),
    )(x, W1, degp)


def _p1b_body(e2_ref, degp_ref, es_ref):
    dinv = _dinv_of(degp_ref)
    es_ref[...] = jnp.concatenate(
        [e2_ref[...] * dinv, jnp.zeros((ROWS, 64), jnp.float32)], axis=1
    )


def _p1b(e2, degp):
    return pl.pallas_call(
        _p1b_body,
        grid=(N // ROWS,),
        in_specs=[
            pl.BlockSpec((ROWS, 64), lambda i: (i, 0)),
            pl.BlockSpec((2, ROWS, 16), lambda i: (0, i, 0)),
        ],
        out_specs=pl.BlockSpec((ROWS, 128), lambda i: (i, 0)),
        out_shape=jax.ShapeDtypeStruct((N, 128), jnp.float32),
    )(e2, degp)


def _p4_body(rx_ref, re_ref, hxs_ref, es_ref, degp_ref,
             b1_ref, ber2_ref, web_ref, w4_ref, hm2_ref, sums_ref, snr_ref):
    dinv = _dinv_of(degp_ref)
    hx = jnp.maximum((rx_ref[0] + rx_ref[1] + hxs_ref[...]) * dinv + b1_ref[...], 0.0)
    agge = ((re_ref[0] + re_ref[1] + es_ref[...]) * dinv)[:, :64]
    hecat = (
        jnp.dot(agge, web_ref[...], preferred_element_type=jnp.float32)
        + ber2_ref[...]
    )
    he0 = hecat[:, :128]
    he1 = hecat[:, 128:]
    hcat = jnp.concatenate([hx + he0, hx + he1], axis=1)
    hm2_ref[...] = (
        jnp.dot(hcat, w4_ref[...], preferred_element_type=jnp.float32) * dinv
    )
    part = jnp.stack(
        [
            jnp.sum(hx * hx, axis=0),
            jnp.sum(he0 * he0, axis=0),
            jnp.sum(he1 * he1, axis=0),
        ]
    )

    @pl.when(pl.program_id(0) == 0)
    def _init():
        sums_ref[...] = jnp.zeros_like(sums_ref)

    sums_ref[...] += part

    @pl.when(pl.program_id(0) == pl.num_programs(0) - 1)
    def _snr():
        t = jnp.sum(sums_ref[...], axis=1)
        snr_ref[...] = jnp.broadcast_to(
            (t[0] / jnp.stack([t[1], t[2]]))[:, None], (2, 128)
        )


def _p4(rx, re, hxs, es, degp, b1r, ber2, web, w4):
    return pl.pallas_call(
        _p4_body,
        grid=(N // ROWS,),
        in_specs=[
            pl.BlockSpec((2, ROWS, 128), lambda i: (0, i, 0)),
            pl.BlockSpec((2, ROWS, 128), lambda i: (0, i, 0)),
            pl.BlockSpec((ROWS, 128), lambda i: (i, 0)),
            pl.BlockSpec((ROWS, 128), lambda i: (i, 0)),
            pl.BlockSpec((2, ROWS, 16), lambda i: (0, i, 0)),
            pl.BlockSpec((1, HID), lambda i: (0, 0)),
            pl.BlockSpec((1, 256), lambda i: (0, 0)),
            pl.BlockSpec((64, 256), lambda i: (0, 0)),
            pl.BlockSpec((256, 128), lambda i: (0, 0)),
        ],
        out_specs=[
            pl.BlockSpec((ROWS, 128), lambda i: (i, 0)),
            pl.BlockSpec((3, 128), lambda i: (0, 0)),
            pl.BlockSpec((2, 128), lambda i: (0, 0)),
        ],
        out_shape=[
            jax.ShapeDtypeStruct((N, 128), jnp.float32),
            jax.ShapeDtypeStruct((3, 128), jnp.float32),
            jax.ShapeDtypeStruct((2, 128), jnp.float32),
        ],
    )(rx, re, hxs, es, degp, b1r, ber2, web, w4)


def _p6_body(r2_ref, hm2_ref, degp_ref, bms_ref, eps_ref, rk_ref,
             mu_ref, sig_ref, z_ref, rk2_ref):
    dinv = _dinv_of(degp_ref)
    ms = (r2_ref[0] + r2_ref[1] + hm2_ref[...]) * dinv + bms_ref[...]
    mu_ref[0] = ms[:, :32]
    mu_ref[1] = ms[:, 32:64]
    sig_ref[0] = ms[:, 64:96]
    sig_ref[1] = ms[:, 96:128]
    z_ref[...] = eps_ref[...] * jnp.exp(ms[:, 96:128] * 0.5) + ms[:, 32:64]

    @pl.when(pl.program_id(0) == 0)
    def _rk():
        rk2_ref[...] = jax.nn.sigmoid(rk_ref[...])


def _p6(r2, hm2, degp, bms, eps0, rk_lgt):
    return pl.pallas_call(
        _p6_body,
        grid=(N // ROWS,),
        in_specs=[
            pl.BlockSpec((2, ROWS, 128), lambda i: (0, i, 0)),
            pl.BlockSpec((ROWS, 128), lambda i: (i, 0)),
            pl.BlockSpec((2, ROWS, 16), lambda i: (0, i, 0)),
            pl.BlockSpec((1, 128), lambda i: (0, 0)),
            pl.BlockSpec((ROWS, OUT), lambda i: (i, 0)),
            pl.BlockSpec((1, OUT), lambda i: (0, 0)),
        ],
        out_specs=[
            pl.BlockSpec((2, ROWS, OUT), lambda i: (0, i, 0)),
            pl.BlockSpec((2, ROWS, OUT), lambda i: (0, i, 0)),
            pl.BlockSpec((ROWS, OUT), lambda i: (i, 0)),
            pl.BlockSpec((1, OUT), lambda i: (0, 0)),
        ],
        out_shape=[
            jax.ShapeDtypeStruct((2, N, OUT), jnp.float32),
            jax.ShapeDtypeStruct((2, N, OUT), jnp.float32),
            jax.ShapeDtypeStruct((N, OUT), jnp.float32),
            jax.ShapeDtypeStruct((1, OUT), jnp.float32),
        ],
    )(r2, hm2, degp, bms, eps0, rk_lgt)


def _p7_body(zb_ref, zf_ref, adj_ref):
    prod = lax.dot_general(
        zb_ref[...], zf_ref[...], (((1,), (1,)), ((), ())),
        preferred_element_type=jnp.float32,
    )
    adj_ref[...] = jax.nn.sigmoid(prod)


def _p7(z):
    return pl.pallas_call(
        _p7_body,
        grid=(N // ROWS,),
        in_specs=[
            pl.BlockSpec((ROWS, OUT), lambda i: (i, 0)),
            pl.BlockSpec((N, OUT), lambda i: (0, 0)),
        ],
        out_specs=pl.BlockSpec((ROWS, N), lambda i: (i, 0)),
        out_shape=jax.ShapeDtypeStruct((N, N), jnp.float32),
    )(z, z)


def kernel(x, edge_index, W1, b1, We, be, Wmu, bmu, Wsig, bsig, rk_lgt):
    f32 = jnp.float32
    src = edge_index[0].astype(jnp.int32).reshape(E // EPB, EPB)
    dst = edge_index[1].astype(jnp.int32).reshape(E // EPB, EPB)

    e = jax.random.normal(jax.random.key(42), (2, N, E_CH), f32) * REWEIGHT
    eps = jax.random.normal(jax.random.key(7), (1, N, OUT), f32)
    e2 = jnp.concatenate([e[0], e[1]], axis=1)
    web = (
        jnp.zeros((64, 256), f32).at[:32, :128].set(We).at[32:, 128:].set(We)
    )
    w4 = (
        jnp.zeros((256, 128), f32)
        .at[:128, :32].set(Wmu)
        .at[128:, 32:64].set(Wmu)
        .at[:128, 64:96].set(Wsig)
        .at[128:, 96:].set(Wsig)
    )
    bms = jnp.concatenate([bmu, bmu, bsig, bsig]).reshape(1, 128)
    b1r = b1.reshape(1, HID)
    ber2 = jnp.concatenate([be, be]).reshape(1, 256)

    degp = _hist_k(dst)
    hxs = _p1a(x, W1, degp)
    (rx,) = _agg1(src, dst, hxs)
    es = _p1b(e2, degp)
    (re,) = _agg1(src, dst, es)
    hm2, sums, snr2 = _p4(rx, re, hxs, es, degp, b1r, ber2, web, w4)
    (r2,) = _agg1(src, dst, hm2)
    mu, sigma, z, rk2 = _p6(r2, hm2, degp, bms, eps[0], rk_lgt)
    adj = _p7(z)

    snr = snr2[:, 0]
    z3 = z[None]
    return adj[None], mu, sigma, z3, z3, eps, rk2, snr


# speedup vs baseline: 122.8150x; 1.0012x over previous
import functools

import jax
import jax.numpy as jnp
from jax import lax
from jax.experimental import pallas as pl
from jax.experimental.pallas import tpu as pltpu
from jax.experimental.pallas import tpu_sc as plsc

N = 4096
E = 65536
IN_CH = 4096
HID = 128
OUT = 32
E_CH = 32
REWEIGHT = ((E_CH + HID) / (IN_CH + HID)) ** 0.5

ROWS = 512
EPB = 128
BLK = 16


def _mesh():
    return plsc.VectorSubcoreMesh(
        core_axis_name="c", subcore_axis_name="s", num_cores=2, num_subcores=16
    )


def _fill_rows(ref, rows, cols, val):
    vec = jnp.full((16,), val, jnp.float32)

    def body(i, carry):
        for k in range(cols // 16):
            ref[i, pl.ds(k * 16, 16)] = vec
        return carry

    lax.fori_loop(0, rows, body, 0)


@functools.partial(
    pl.kernel,
    out_type=jax.ShapeDtypeStruct((2, N, 16), jnp.float32),
    mesh=_mesh(),
    scratch_types=[
        pltpu.VMEM((BLK, EPB), jnp.int32),
        pltpu.VMEM((EPB, 16), jnp.float32),
        pltpu.VMEM((256, 16), jnp.float32),
        pltpu.VMEM_SHARED((N, 16), jnp.float32),
    ],
)
def _hist_k(dst_hbm, out_hbm, idx_v, ones_v, zbuf, acc):
    c = lax.axis_index("c")
    s = lax.axis_index("s")
    w = s * 2 + c
    _fill_rows(zbuf, 256, 16, 0.0)
    _fill_rows(ones_v, EPB, 16, 1.0)
    pltpu.sync_copy(zbuf, acc.at[pl.ds(s * 256, 256)])
    plsc.subcore_barrier()
    pltpu.sync_copy(dst_hbm.at[pl.ds(w * BLK, BLK)], idx_v)
    for j in range(BLK):
        pltpu.sync_copy(ones_v, acc.at[idx_v.at[j]], add=True)
    plsc.subcore_barrier()
    pltpu.sync_copy(acc.at[pl.ds(s * 256, 256)], out_hbm.at[c, pl.ds(s * 256, 256)])


def _make_agg(n_feats):
    out_types = [jax.ShapeDtypeStruct((2, N, 128), jnp.float32)] * n_feats

    @functools.partial(
        pl.kernel,
        out_type=out_types,
        mesh=_mesh(),
        scratch_types=[
            pltpu.VMEM((BLK, EPB), jnp.int32),
            pltpu.VMEM((BLK, EPB), jnp.int32),
            pltpu.VMEM((4, EPB, 128), jnp.float32),
            pltpu.VMEM((128, 128), jnp.float32),
            pltpu.VMEM_SHARED((N, 128), jnp.float32),
        ] + [pltpu.SemaphoreType.DMA] * 8,
    )
    def agg_k(src_hbm, dst_hbm, *rest):
        feats = rest[:n_feats]
        outs = rest[n_feats:2 * n_feats]
        src_v, dst_v, gbuf, zbuf, acc = rest[2 * n_feats:2 * n_feats + 5]
        gsems = rest[2 * n_feats + 5:2 * n_feats + 9]
        ssems = rest[2 * n_feats + 9:2 * n_feats + 13]
        c = lax.axis_index("c")
        s = lax.axis_index("s")
        w = s * 2 + c
        K = 4
        LA = 2

        _fill_rows(zbuf, 128, 128, 0.0)
        pltpu.sync_copy(zbuf, acc.at[pl.ds(s * 256, 128)])
        pltpu.sync_copy(zbuf, acc.at[pl.ds(s * 256 + 128, 128)])
        pltpu.sync_copy(src_hbm.at[pl.ds(w * BLK, BLK)], src_v)
        pltpu.sync_copy(dst_hbm.at[pl.ds(w * BLK, BLK)], dst_v)
        plsc.subcore_barrier()

        for f in range(n_feats):
            feat = feats[f]
            gd = [None] * K
            sd = [None] * K
            for j in range(LA):
                gd[j % K] = pltpu.async_copy(
                    feat.at[src_v.at[j]], gbuf.at[j % K], gsems[j % K]
                )
            for j in range(BLK):
                b = j % K
                gd[b].wait()
                sd[b] = pltpu.async_copy(
                    gbuf.at[b], acc.at[dst_v.at[j]], ssems[b], add=True
                )
                jg = j + LA
                if jg < BLK:
                    bg = jg % K
                    if sd[bg] is not None:
                        sd[bg].wait()
                        sd[bg] = None
                    gd[bg] = pltpu.async_copy(
                        feat.at[src_v.at[jg]], gbuf.at[bg], gsems[bg]
                    )
            for b in range(K):
                if sd[b] is not None:
                    sd[b].wait()
            plsc.subcore_barrier()
            out = outs[f]
            pltpu.sync_copy(acc.at[pl.ds(s * 256, 128)], out.at[c, pl.ds(s * 256, 128)])
            pltpu.sync_copy(
                acc.at[pl.ds(s * 256 + 128, 128)], out.at[c, pl.ds(s * 256 + 128, 128)]
            )
            if f + 1 < n_feats:
                pltpu.sync_copy(zbuf, acc.at[pl.ds(s * 256, 128)])
                pltpu.sync_copy(zbuf, acc.at[pl.ds(s * 256 + 128, 128)])
                plsc.subcore_barrier()

    return agg_k


_agg1 = _make_agg(1)


def _dinv_of(degp_ref):
    deg = degp_ref[0] + degp_ref[1]
    return lax.rsqrt(deg[:, :1] + 1.0)


def _p1a_body(x_ref, w1_ref, degp_ref, hx_ref):
    dinv = _dinv_of(degp_ref)
    h1 = jnp.dot(x_ref[...], w1_ref[...], preferred_element_type=jnp.float32)
    hx_ref[...] = h1 * dinv


def _p1a(x, W1, degp):
    return pl.pallas_call(
        _p1a_body,
        grid=(N // ROWS,),
        in_specs=[
            pl.BlockSpec((ROWS, IN_CH), lambda i: (i, 0)),
            pl.BlockSpec((IN_CH, HID), lambda i: (0, 0)),
            pl.BlockSpec((2, ROWS, 16), lambda i: (0, i, 0)),
        ],
        out_specs=pl.BlockSpec((ROWS, 128), lambda i: (i, 0)),
        out_shape=jax.ShapeDtypeStruct((N, 128), jnp.float32),
    )(x, W1, degp)


def _p1b_body(e2_ref, degp_ref, es_ref):
    dinv = _dinv_of(degp_ref)
    es_ref[...] = jnp.concatenate(
        [e2_ref[...] * dinv, jnp.zeros((ROWS, 64), jnp.float32)], axis=1
    )


def _p1b(e2, degp):
    return pl.pallas_call(
        _p1b_body,
        grid=(N // ROWS,),
        in_specs=[
            pl.BlockSpec((ROWS, 64), lambda i: (i, 0)),
            pl.BlockSpec((2, ROWS, 16), lambda i: (0, i, 0)),
        ],
        out_specs=pl.BlockSpec((ROWS, 128), lambda i: (i, 0)),
        out_shape=jax.ShapeDtypeStruct((N, 128), jnp.float32),
    )(e2, degp)


def _p4_body(rx_ref, re_ref, hxs_ref, es_ref, degp_ref,
             b1_ref, ber2_ref, web_ref, w4_ref, hm2_ref, sums_ref, snr_ref):
    dinv = _dinv_of(degp_ref)
    hx = jnp.maximum((rx_ref[0] + rx_ref[1] + hxs_ref[...]) * dinv + b1_ref[...], 0.0)
    agge = ((re_ref[0] + re_ref[1] + es_ref[...]) * dinv)[:, :64]
    hecat = (
        jnp.dot(agge, web_ref[...], preferred_element_type=jnp.float32)
        + ber2_ref[...]
    )
    he0 = hecat[:, :128]
    he1 = hecat[:, 128:]
    hcat = jnp.concatenate([hx + he0, hx + he1], axis=1)
    hm2_ref[...] = (
        jnp.dot(hcat, w4_ref[...], preferred_element_type=jnp.float32) * dinv
    )
    part = jnp.stack(
        [
            jnp.sum(hx * hx, axis=0),
            jnp.sum(he0 * he0, axis=0),
            jnp.sum(he1 * he1, axis=0),
        ]
    )

    @pl.when(pl.program_id(0) == 0)
    def _init():
        sums_ref[...] = jnp.zeros_like(sums_ref)

    sums_ref[...] += part

    @pl.when(pl.program_id(0) == pl.num_programs(0) - 1)
    def _snr():
        t = jnp.sum(sums_ref[...], axis=1)
        snr_ref[...] = jnp.broadcast_to(
            (t[0] / jnp.stack([t[1], t[2]]))[:, None], (2, 128)
        )


def _p4(rx, re, hxs, es, degp, b1r, ber2, web, w4):
    return pl.pallas_call(
        _p4_body,
        grid=(N // ROWS,),
        in_specs=[
            pl.BlockSpec((2, ROWS, 128), lambda i: (0, i, 0)),
            pl.BlockSpec((2, ROWS, 128), lambda i: (0, i, 0)),
            pl.BlockSpec((ROWS, 128), lambda i: (i, 0)),
            pl.BlockSpec((ROWS, 128), lambda i: (i, 0)),
            pl.BlockSpec((2, ROWS, 16), lambda i: (0, i, 0)),
            pl.BlockSpec((1, HID), lambda i: (0, 0)),
            pl.BlockSpec((1, 256), lambda i: (0, 0)),
            pl.BlockSpec((64, 256), lambda i: (0, 0)),
            pl.BlockSpec((256, 128), lambda i: (0, 0)),
        ],
        out_specs=[
            pl.BlockSpec((ROWS, 128), lambda i: (i, 0)),
            pl.BlockSpec((3, 128), lambda i: (0, 0)),
            pl.BlockSpec((2, 128), lambda i: (0, 0)),
        ],
        out_shape=[
            jax.ShapeDtypeStruct((N, 128), jnp.float32),
            jax.ShapeDtypeStruct((3, 128), jnp.float32),
            jax.ShapeDtypeStruct((2, 128), jnp.float32),
        ],
    )(rx, re, hxs, es, degp, b1r, ber2, web, w4)


def _p6_body(r2_ref, hm2_ref, degp_ref, bms_ref, eps_ref, rk_ref,
             mu_ref, sig_ref, z_ref, rk2_ref):
    dinv = _dinv_of(degp_ref)
    ms = (r2_ref[0] + r2_ref[1] + hm2_ref[...]) * dinv + bms_ref[...]
    mu_ref[0] = ms[:, :32]
    mu_ref[1] = ms[:, 32:64]
    sig_ref[0] = ms[:, 64:96]
    sig_ref[1] = ms[:, 96:128]
    z_ref[...] = eps_ref[...] * jnp.exp(ms[:, 96:128] * 0.5) + ms[:, 32:64]

    @pl.when(pl.program_id(0) == 0)
    def _rk():
        rk2_ref[...] = jax.nn.sigmoid(rk_ref[...])


def _p6(r2, hm2, degp, bms, eps0, rk_lgt):
    return pl.pallas_call(
        _p6_body,
        grid=(N // ROWS,),
        in_specs=[
            pl.BlockSpec((2, ROWS, 128), lambda i: (0, i, 0)),
            pl.BlockSpec((ROWS, 128), lambda i: (i, 0)),
            pl.BlockSpec((2, ROWS, 16), lambda i: (0, i, 0)),
            pl.BlockSpec((1, 128), lambda i: (0, 0)),
            pl.BlockSpec((ROWS, OUT), lambda i: (i, 0)),
            pl.BlockSpec((1, OUT), lambda i: (0, 0)),
        ],
        out_specs=[
            pl.BlockSpec((2, ROWS, OUT), lambda i: (0, i, 0)),
            pl.BlockSpec((2, ROWS, OUT), lambda i: (0, i, 0)),
            pl.BlockSpec((ROWS, OUT), lambda i: (i, 0)),
            pl.BlockSpec((1, OUT), lambda i: (0, 0)),
        ],
        out_shape=[
            jax.ShapeDtypeStruct((2, N, OUT), jnp.float32),
            jax.ShapeDtypeStruct((2, N, OUT), jnp.float32),
            jax.ShapeDtypeStruct((N, OUT), jnp.float32),
            jax.ShapeDtypeStruct((1, OUT), jnp.float32),
        ],
    )(r2, hm2, degp, bms, eps0, rk_lgt)


def _p7_body(zb_ref, zf_ref, adj_ref):
    prod = lax.dot_general(
        zb_ref[...], zf_ref[...], (((1,), (1,)), ((), ())),
        preferred_element_type=jnp.float32,
    )
    adj_ref[...] = jax.nn.sigmoid(prod)


def _p7(z):
    return pl.pallas_call(
        _p7_body,
        grid=(N // ROWS,),
        in_specs=[
            pl.BlockSpec((ROWS, OUT), lambda i: (i, 0)),
            pl.BlockSpec((N, OUT), lambda i: (0, 0)),
        ],
        out_specs=pl.BlockSpec((ROWS, N), lambda i: (i, 0)),
        out_shape=jax.ShapeDtypeStruct((N, N), jnp.float32),
    )(z, z)


def kernel(x, edge_index, W1, b1, We, be, Wmu, bmu, Wsig, bsig, rk_lgt):
    f32 = jnp.float32
    src = edge_index[0].astype(jnp.int32).reshape(E // EPB, EPB)
    dst = edge_index[1].astype(jnp.int32).reshape(E // EPB, EPB)

    e = jax.random.normal(jax.random.key(42), (2, N, E_CH), f32) * REWEIGHT
    eps = jax.random.normal(jax.random.key(7), (1, N, OUT), f32)
    e2 = jnp.concatenate([e[0], e[1]], axis=1)
    web = (
        jnp.zeros((64, 256), f32).at[:32, :128].set(We).at[32:, 128:].set(We)
    )
    w4 = (
        jnp.zeros((256, 128), f32)
        .at[:128, :32].set(Wmu)
        .at[128:, 32:64].set(Wmu)
        .at[:128, 64:96].set(Wsig)
        .at[128:, 96:].set(Wsig)
    )
    bms = jnp.concatenate([bmu, bmu, bsig, bsig]).reshape(1, 128)
    b1r = b1.reshape(1, HID)
    ber2 = jnp.concatenate([be, be]).reshape(1, 256)

    degp = _hist_k(dst)
    hxs = _p1a(x, W1, degp)
    (rx,) = _agg1(src, dst, hxs)
    es = _p1b(e2, degp)
    (re,) = _agg1(src, dst, es)
    hm2, sums, snr2 = _p4(rx, re, hxs, es, degp, b1r, ber2, web, w4)
    (r2,) = _agg1(src, dst, hm2)
    mu, sigma, z, rk2 = _p6(r2, hm2, degp, bms, eps[0], rk_lgt)
    adj = _p7(z)

    snr = snr2[:, 0]
    z3 = z[None]
    return adj[None], mu, sigma, z3, z3, eps, rk2, snr
